# baseline (speedup 1.0000x reference)
import functools

import jax
import jax.numpy as jnp
from jax import lax
from jax.experimental import pallas as pl
from jax.experimental.pallas import tpu as pltpu
from jax.experimental.pallas import tpu_sc as plsc

N_NODES = 10000
N_EDGES = 320000
D = 128
NC = 2
NS = 16
NW = NC * NS
EPT = N_EDGES // NW
CH = 80
NBUF = 3
K = 126
EPP = K * CH
NP = 10240
RPT = NP // NS


def _elu_body(g_ref, w_ref, out_ref):
    x = g_ref[...] * w_ref[...]
    out_ref[...] = jnp.where(x > 0, x, jnp.exp(jnp.minimum(x, 0.0)) - 1.0)


def _add_body(p_ref, out_ref):
    out_ref[...] = p_ref[0, :N_NODES] + p_ref[1, :N_NODES]


def _sc_body(src_hbm, dst_hbm, emb_hbm, out_hbm, src_v, rows, dstb, acc,
             gsem, dsem):
    cid = lax.axis_index("c")
    sid = lax.axis_index("s")
    wid = cid * NS + sid

    def zstore(t, carry):
        i = t // (D // 16)
        j = t % (D // 16)
        rows[0][i, pl.ds(j * 16, 16)] = jnp.zeros((16,), jnp.float32)
        return carry

    lax.fori_loop(0, CH * (D // 16), zstore, 0)

    row0 = sid * RPT
    for r in range(RPT // CH):
        pltpu.sync_copy(rows[0], acc.at[pl.ds(row0 + r * CH, CH)])
    plsc.subcore_barrier()

    pltpu.sync_copy(src_hbm.at[wid], src_v)
    dbase = wid * EPP

    for b in range(NBUF):
        pltpu.async_copy(dst_hbm.at[pl.ds(dbase + b * CH, CH)], dstb[b],
                         dsem[b])
        pltpu.async_copy(emb_hbm.at[src_v.at[b]], rows[b], gsem[b])

    def round_(g, carry):
        j0 = NBUF * g
        for b in range(NBUF):
            j = j0 + b
            pltpu.make_async_copy(emb_hbm.at[pl.ds(0, CH)], rows[b],
                                  gsem[b]).wait()
            pltpu.make_async_copy(dst_hbm.at[pl.ds(0, CH)], dstb[b],
                                  dsem[b]).wait()
            pltpu.sync_copy(rows[b], acc.at[dstb[b]], add=True)

            @pl.when(j + NBUF < K)
            def _():
                pltpu.async_copy(
                    dst_hbm.at[pl.ds(dbase + (j + NBUF) * CH, CH)], dstb[b],
                    dsem[b])
                pltpu.async_copy(emb_hbm.at[src_v.at[j + NBUF]], rows[b],
                                 gsem[b])

        return carry

    lax.fori_loop(0, K // NBUF, round_, 0)

    plsc.subcore_barrier()
    pltpu.sync_copy(acc.at[pl.ds(row0, RPT)],
                    out_hbm.at[cid, pl.ds(row0, RPT)])


_sc_scatter = functools.partial(
    pl.kernel,
    out_type=jax.ShapeDtypeStruct((NC, NP, D), jnp.float32),
    mesh=plsc.VectorSubcoreMesh(core_axis_name="c", subcore_axis_name="s"),
    scratch_types=[
        pltpu.VMEM((K, CH), jnp.int32),
        [pltpu.VMEM((CH, D), jnp.float32) for _ in range(NBUF)],
        [pltpu.VMEM((CH,), jnp.int32) for _ in range(NBUF)],
        pltpu.VMEM_SHARED((NP, D), jnp.float32),
        [pltpu.SemaphoreType.DMA for _ in range(NBUF)],
        [pltpu.SemaphoreType.DMA for _ in range(NBUF)],
    ],
)(_sc_body)


def kernel(edge_index, graph_embedding, weight):
    ei = edge_index.astype(jnp.int32)
    src2 = ei[0].reshape(NW, EPT)
    dst2 = ei[1].reshape(NW, EPT)
    pad = EPP - EPT
    src = jnp.pad(src2, ((0, 0), (0, pad))).reshape(NW, K, CH)
    dump = N_NODES + jnp.arange(NW, dtype=jnp.int32)
    dst = jnp.concatenate(
        [dst2, jnp.broadcast_to(dump[:, None], (NW, pad))], axis=1
    ).reshape(NW * EPP)

    emb = pl.pallas_call(
        _elu_body,
        out_shape=jax.ShapeDtypeStruct((N_NODES, D), jnp.float32),
    )(graph_embedding, weight)

    partials = _sc_scatter(src, dst, emb)

    out = pl.pallas_call(
        _add_body,
        out_shape=jax.ShapeDtypeStruct((N_NODES, D), jnp.float32),
    )(partials)
    return out
